# baseline (speedup 1.0000x reference)
import jax
import jax.numpy as jnp
from jax.experimental import pallas as pl
from jax.experimental.pallas import tpu as pltpu

ALPHA = 0.25
W_CLASS = 2.0
W_BBOX = 5.0
W_GIOU = 2.0
EPS_LOG = 1e-8
EPS_DIV = 1e-6

_QCHUNK = 128
_TSLICE = 256


def _cost_kernel(logits_ref, pb_ref, tb_ref, tid_ref, out_ref):
    q = out_ref.shape[1]
    t = out_ref.shape[2]

    idm1 = tid_ref[0:1, :] - 1
    cxt = tb_ref[0:1, :]
    cyt = tb_ref[1:2, :]
    wt = tb_ref[2:3, :]
    ht = tb_ref[3:4, :]
    cxt5 = W_BBOX * cxt
    cyt5 = W_BBOX * cyt
    wt5 = W_BBOX * wt
    ht5 = W_BBOX * ht
    x0t = cxt - 0.5 * wt
    y0t = cyt - 0.5 * ht
    x1t = cxt + 0.5 * wt
    y1t = cyt + 0.5 * ht
    areat = (x1t - x0t) * (y1t - y0t)

    def do_chunk(rows, m):
        s = jax.nn.sigmoid(logits_ref[0, rows, :])
        one_m = 1.0 - s
        neg = (1.0 - ALPHA) * (s * s) * (-jnp.log(one_m + EPS_LOG))
        pos = ALPHA * (one_m * one_m) * (-jnp.log(s + EPS_LOG))
        diff = W_CLASS * (pos - neg)

        cxq = pb_ref[0, rows, 0:1]
        cyq = pb_ref[0, rows, 1:2]
        wq = pb_ref[0, rows, 2:3]
        hq = pb_ref[0, rows, 3:4]
        cxq5 = W_BBOX * cxq
        cyq5 = W_BBOX * cyq
        wq5 = W_BBOX * wq
        hq5 = W_BBOX * hq
        x0q = cxq - 0.5 * wq
        y0q = cyq - 0.5 * hq
        x1q = cxq + 0.5 * wq
        y1q = cyq + 0.5 * hq
        areaq = (x1q - x0q) * (y1q - y0q)

        for c0 in range(0, t, _TSLICE):
            c1 = min(c0 + _TSLICE, t)
            cols = slice(c0, c1)

            idx = jnp.broadcast_to(idm1[:, cols], (m, c1 - c0))
            cost_class = jnp.take_along_axis(diff, idx, axis=1)

            bbox5 = (jnp.abs(cxq5 - cxt5[:, cols])
                     + jnp.abs(cyq5 - cyt5[:, cols])
                     + jnp.abs(wq5 - wt5[:, cols])
                     + jnp.abs(hq5 - ht5[:, cols]))

            x0 = x0t[:, cols]
            y0 = y0t[:, cols]
            x1 = x1t[:, cols]
            y1 = y1t[:, cols]

            iw = jnp.maximum(jnp.minimum(x1q, x1) - jnp.maximum(x0q, x0), 0.0)
            ih = jnp.maximum(jnp.minimum(y1q, y1) - jnp.maximum(y0q, y0), 0.0)
            inter = iw * ih
            union = areaq + areat[:, cols] - inter
            iou = inter / jnp.maximum(union, EPS_DIV)
            ew = jnp.maximum(x1q, x1) - jnp.minimum(x0q, x0)
            eh = jnp.maximum(y1q, y1) - jnp.minimum(y0q, y0)
            encl = ew * eh
            giou = iou - (encl - union) / jnp.maximum(encl, EPS_DIV)

            out_ref[0, rows, cols] = (cost_class + bbox5
                                      - W_GIOU * giou)

    n_full = q // _QCHUNK

    def body(i, carry):
        a = pl.multiple_of(i * _QCHUNK, _QCHUNK)
        do_chunk(pl.ds(a, _QCHUNK), _QCHUNK)
        return carry

    jax.lax.fori_loop(0, n_full, body, 0)
    if q % _QCHUNK:
        do_chunk(slice(n_full * _QCHUNK, q), q - n_full * _QCHUNK)


def kernel(pred_logits, pred_boxes, tgt_ids, tgt_boxes):
    B, Q, C = pred_logits.shape
    T = tgt_ids.shape[0]

    logits = jnp.pad(pred_logits, ((0, 0), (0, 0), (0, 128 - C)))
    tb = jnp.pad(tgt_boxes.T, ((0, 4), (0, 0)))
    tid = tgt_ids.astype(jnp.int32).reshape(1, T)

    return pl.pallas_call(
        _cost_kernel,
        out_shape=jax.ShapeDtypeStruct((B, Q, T), jnp.float32),
        grid=(B,),
        in_specs=[
            pl.BlockSpec((1, Q, 128), lambda i: (i, 0, 0)),
            pl.BlockSpec((1, Q, 4), lambda i: (i, 0, 0)),
            pl.BlockSpec((8, T), lambda i: (0, 0)),
            pl.BlockSpec((1, T), lambda i: (0, 0)),
        ],
        out_specs=pl.BlockSpec((1, Q, T), lambda i: (i, 0, 0)),
        compiler_params=pltpu.CompilerParams(
            dimension_semantics=("arbitrary",),
            vmem_limit_bytes=56 * 1024 * 1024,
        ),
        name="hungarian_cost_matrix",
    )(logits, pred_boxes, tb, tid)


# speedup vs baseline: 1.0775x; 1.0775x over previous
import jax
import jax.numpy as jnp
from jax.experimental import pallas as pl
from jax.experimental.pallas import tpu as pltpu

ALPHA = 0.25
W_CLASS = 2.0
W_BBOX = 5.0
W_GIOU = 2.0
EPS_LOG = 1e-8
EPS_DIV = 1e-6

_QCHUNK = 128
_TSLICE = 128


def _cost_kernel(logits_ref, pb_ref, tb_ref, tid_ref, out_ref):
    q = out_ref.shape[1]
    t = out_ref.shape[2]

    idm1 = tid_ref[0:1, :] - 1
    cxt = tb_ref[0:1, :]
    cyt = tb_ref[1:2, :]
    wt = tb_ref[2:3, :]
    ht = tb_ref[3:4, :]
    bf = jnp.bfloat16
    cxt5 = (W_BBOX * cxt).astype(bf)
    cyt5 = (W_BBOX * cyt).astype(bf)
    wt5 = (W_BBOX * wt).astype(bf)
    ht5 = (W_BBOX * ht).astype(bf)
    x0t = cxt - 0.5 * wt
    y0t = cyt - 0.5 * ht
    x1t = cxt + 0.5 * wt
    y1t = cyt + 0.5 * ht
    areat = (x1t - x0t) * (y1t - y0t)

    def do_chunk(rows, m):
        s = jax.nn.sigmoid(logits_ref[0, rows, :])
        one_m = 1.0 - s
        neg = (1.0 - ALPHA) * (s * s) * (-jnp.log(one_m + EPS_LOG))
        pos = ALPHA * (one_m * one_m) * (-jnp.log(s + EPS_LOG))
        diff = W_CLASS * (pos - neg)

        cxq = pb_ref[0, rows, 0:1]
        cyq = pb_ref[0, rows, 1:2]
        wq = pb_ref[0, rows, 2:3]
        hq = pb_ref[0, rows, 3:4]
        bf = jnp.bfloat16
        cxq5 = (W_BBOX * cxq).astype(bf)
        cyq5 = (W_BBOX * cyq).astype(bf)
        wq5 = (W_BBOX * wq).astype(bf)
        hq5 = (W_BBOX * hq).astype(bf)
        x0q = cxq - 0.5 * wq
        y0q = cyq - 0.5 * hq
        x1q = cxq + 0.5 * wq
        y1q = cyq + 0.5 * hq
        areaq = (x1q - x0q) * (y1q - y0q)

        for c0 in range(0, t, _TSLICE):
            c1 = min(c0 + _TSLICE, t)
            cols = slice(c0, c1)

            idx = jnp.broadcast_to(idm1[:, cols], (m, c1 - c0))
            cost_class = jnp.take_along_axis(diff, idx, axis=1)

            bbox5 = (jnp.abs(cxq5 - cxt5[:, cols])
                     + jnp.abs(cyq5 - cyt5[:, cols])
                     + jnp.abs(wq5 - wt5[:, cols])
                     + jnp.abs(hq5 - ht5[:, cols])).astype(jnp.float32)

            x0 = x0t[:, cols]
            y0 = y0t[:, cols]
            x1 = x1t[:, cols]
            y1 = y1t[:, cols]

            iw = jnp.maximum(jnp.minimum(x1q, x1) - jnp.maximum(x0q, x0), 0.0)
            ih = jnp.maximum(jnp.minimum(y1q, y1) - jnp.maximum(y0q, y0), 0.0)
            inter = iw * ih
            union = areaq + areat[:, cols] - inter
            iou = inter / jnp.maximum(union, EPS_DIV)
            ew = jnp.maximum(x1q, x1) - jnp.minimum(x0q, x0)
            eh = jnp.maximum(y1q, y1) - jnp.minimum(y0q, y0)
            encl = ew * eh
            giou = iou - (encl - union) / jnp.maximum(encl, EPS_DIV)

            out_ref[0, rows, cols] = (cost_class + bbox5
                                      - W_GIOU * giou)

    n_full = q // _QCHUNK

    def body(i, carry):
        a = pl.multiple_of(i * _QCHUNK, _QCHUNK)
        do_chunk(pl.ds(a, _QCHUNK), _QCHUNK)
        return carry

    jax.lax.fori_loop(0, n_full, body, 0)
    if q % _QCHUNK:
        do_chunk(slice(n_full * _QCHUNK, q), q - n_full * _QCHUNK)


def kernel(pred_logits, pred_boxes, tgt_ids, tgt_boxes):
    B, Q, C = pred_logits.shape
    T = tgt_ids.shape[0]

    logits = jnp.pad(pred_logits, ((0, 0), (0, 0), (0, 128 - C)))
    tb = jnp.pad(tgt_boxes.T, ((0, 4), (0, 0)))
    tid = tgt_ids.astype(jnp.int32).reshape(1, T)

    return pl.pallas_call(
        _cost_kernel,
        out_shape=jax.ShapeDtypeStruct((B, Q, T), jnp.float32),
        grid=(B,),
        in_specs=[
            pl.BlockSpec((1, Q, 128), lambda i: (i, 0, 0)),
            pl.BlockSpec((1, Q, 4), lambda i: (i, 0, 0)),
            pl.BlockSpec((8, T), lambda i: (0, 0)),
            pl.BlockSpec((1, T), lambda i: (0, 0)),
        ],
        out_specs=pl.BlockSpec((1, Q, T), lambda i: (i, 0, 0)),
        compiler_params=pltpu.CompilerParams(
            dimension_semantics=("arbitrary",),
            vmem_limit_bytes=56 * 1024 * 1024,
        ),
        name="hungarian_cost_matrix",
    )(logits, pred_boxes, tb, tid)
